# baseline (speedup 1.0000x reference)
import functools

import jax
import jax.numpy as jnp
from jax import lax
from jax.experimental import pallas as pl
from jax.experimental.pallas import tpu as pltpu
from jax.experimental.pallas import tpu_sc as plsc

N, D, H, G = 50000, 512, 512, 512
R = 2000
GRID = N // R

TILE_ROWS = 80
N_TILES = N // TILE_ROWS
CBLK = 128
NCB = H // CBLK
NRG = 8
TPW = (N_TILES + NRG - 1) // NRG


def _mlp_body(x_ref, wp_ref, bp_ref, wg_ref, bg_ref, out_ref):
    x = x_ref[...]
    proj = jnp.dot(x, wp_ref[...], preferred_element_type=jnp.float32) + bp_ref[...]
    gate_l = jnp.dot(x, wg_ref[...], preferred_element_type=jnp.float32) + bg_ref[...]
    out_ref[...] = jax.nn.sigmoid(gate_l) * proj


def _gated_mlp(x, wp, bp, wg, bg):
    return pl.pallas_call(
        _mlp_body,
        grid=(GRID,),
        in_specs=[
            pl.BlockSpec((R, D), lambda i: (i, 0)),
            pl.BlockSpec((D, H), lambda i: (0, 0)),
            pl.BlockSpec((1, H), lambda i: (0, 0)),
            pl.BlockSpec((D, H), lambda i: (0, 0)),
            pl.BlockSpec((1, H), lambda i: (0, 0)),
        ],
        out_specs=pl.BlockSpec((R, H), lambda i: (i, 0)),
        out_shape=jax.ShapeDtypeStruct((N, H), jnp.float32),
    )(x, wp, bp, wg, bg)


def _seg_sum_sc(gated, ids):
    mesh = plsc.VectorSubcoreMesh(core_axis_name="c", subcore_axis_name="s")

    @functools.partial(
        pl.kernel,
        mesh=mesh,
        out_type=jax.ShapeDtypeStruct((NRG, G, H), jnp.float32),
        scratch_types=[
            pltpu.VMEM((G, CBLK), jnp.float32),
            pltpu.VMEM((TILE_ROWS, CBLK), jnp.float32),
            pltpu.VMEM((TILE_ROWS,), jnp.int32),
        ],
    )
    def k(gated_hbm, ids_hbm, out_hbm, acc, rows_v, ids_v):
        cid = lax.axis_index("c")
        sid = lax.axis_index("s")
        wid = sid * 2 + cid
        rg = wid // NCB
        col0 = (wid % NCB) * CBLK

        def zr(g, carry):
            for k8 in range(CBLK // 16):
                acc[g, pl.ds(k8 * 16, 16)] = jnp.zeros((16,), jnp.float32)
            return carry

        lax.fori_loop(0, G, zr, 0)

        def tile_body(tt, carry):
            t = tt * NRG + rg

            @pl.when(t < N_TILES)
            def _():
                base = t * TILE_ROWS
                pltpu.sync_copy(ids_hbm.at[pl.ds(base, TILE_ROWS)], ids_v)
                pltpu.sync_copy(
                    gated_hbm.at[pl.ds(base, TILE_ROWS), pl.ds(col0, CBLK)],
                    rows_v)

                def grp(g, c2):
                    idg = ids_v[pl.ds(g * 16, 16)]
                    for j in range(16):
                        sj = idg[j]
                        row = g * 16 + j
                        for k8 in range(CBLK // 16):
                            v = rows_v[row, pl.ds(k8 * 16, 16)]
                            plsc.addupdate(acc.at[sj, pl.ds(k8 * 16, 16)], v)
                    return c2

                lax.fori_loop(0, TILE_ROWS // 16, grp, 0)

            return carry

        lax.fori_loop(0, TPW, tile_body, 0)
        pltpu.sync_copy(acc, out_hbm.at[rg, :, pl.ds(col0, CBLK)])

    return k(gated, ids)


def _add_body(p_ref, o_ref):
    o_ref[...] = jnp.sum(p_ref[...], axis=0)


def _combine(p):
    return pl.pallas_call(
        _add_body,
        out_shape=jax.ShapeDtypeStruct((G, H), jnp.float32),
    )(p)


def kernel(node_features, node_to_graph_id, W_proj, b_proj, W_gate, b_gate):
    ids = node_to_graph_id.astype(jnp.int32)
    gated = _gated_mlp(node_features, W_proj, b_proj.reshape(1, H),
                       W_gate, b_gate.reshape(1, H))
    partials = _seg_sum_sc(gated, ids)
    return _combine(partials)


# speedup vs baseline: 1.4964x; 1.4964x over previous
import functools

import jax
import jax.numpy as jnp
from jax import lax
from jax.experimental import pallas as pl
from jax.experimental.pallas import tpu as pltpu
from jax.experimental.pallas import tpu_sc as plsc

N, D, H, G = 50000, 512, 512, 512
R = 2000
GRID = N // R

TILE_ROWS = 80
N_TILES = N // TILE_ROWS
CBLK = 128
NRG = 8
CPW = N_TILES // NRG
LAST_TILE = CPW * NRG
HC = H // 2


def _mlp_body(x_ref, wp_ref, bp_ref, wg_ref, bg_ref, out_ref):
    x = x_ref[...]
    proj = jnp.dot(x, wp_ref[...], preferred_element_type=jnp.float32) + bp_ref[...]
    gate_l = jnp.dot(x, wg_ref[...], preferred_element_type=jnp.float32) + bg_ref[...]
    out_ref[...] = jax.nn.sigmoid(gate_l) * proj


def _gated_mlp(x, wp, bp, wg, bg):
    return pl.pallas_call(
        _mlp_body,
        grid=(GRID,),
        in_specs=[
            pl.BlockSpec((R, D), lambda i: (i, 0)),
            pl.BlockSpec((D, H), lambda i: (0, 0)),
            pl.BlockSpec((1, H), lambda i: (0, 0)),
            pl.BlockSpec((D, H), lambda i: (0, 0)),
            pl.BlockSpec((1, H), lambda i: (0, 0)),
        ],
        out_specs=pl.BlockSpec((R, H), lambda i: (i, 0)),
        out_shape=jax.ShapeDtypeStruct((N, H), jnp.float32),
    )(x, wp, bp, wg, bg)


def _seg_sum_sc(gated, ids):
    mesh = plsc.VectorSubcoreMesh(core_axis_name="c", subcore_axis_name="s")

    @functools.partial(
        pl.kernel,
        mesh=mesh,
        out_type=(jax.ShapeDtypeStruct((G, H), jnp.float32),
                  jax.ShapeDtypeStruct((NRG, G, H), jnp.float32)),
        scratch_types=[
            pltpu.VMEM((G, CBLK), jnp.float32),
            pltpu.VMEM((TILE_ROWS, CBLK), jnp.float32),
            pltpu.VMEM((TILE_ROWS, CBLK), jnp.float32),
            pltpu.VMEM((TILE_ROWS,), jnp.int32),
            pltpu.VMEM((TILE_ROWS,), jnp.int32),
            pltpu.VMEM((32, HC), jnp.float32),
            pltpu.VMEM((32, HC), jnp.float32),
            pltpu.SemaphoreType.DMA,
            pltpu.SemaphoreType.DMA,
            pltpu.SemaphoreType.DMA,
            pltpu.SemaphoreType.DMA,
        ],
    )
    def k(gated_hbm, ids_hbm, out_hbm, part_hbm, acc, rows0, rows1,
          ids0, ids1, av, pv, sr0, sr1, si0, si1):
        cid = lax.axis_index("c")
        sid = lax.axis_index("s")
        rg = sid // 2
        col0 = cid * HC + (sid % 2) * CBLK

        rows_b = (rows0, rows1)
        ids_b = (ids0, ids1)
        sr = (sr0, sr1)
        si = (si0, si1)

        def start(b, t):
            base = t * TILE_ROWS
            pltpu.async_copy(
                gated_hbm.at[pl.ds(base, TILE_ROWS), pl.ds(col0, CBLK)],
                rows_b[b], sr[b])
            pltpu.async_copy(ids_hbm.at[pl.ds(base, TILE_ROWS)], ids_b[b], si[b])

        def wait(b):
            pltpu.make_async_copy(
                gated_hbm.at[pl.ds(0, TILE_ROWS), pl.ds(col0, CBLK)],
                rows_b[b], sr[b]).wait()
            pltpu.make_async_copy(ids_hbm.at[pl.ds(0, TILE_ROWS)],
                                  ids_b[b], si[b]).wait()

        def compute(b):
            rv, iv = rows_b[b], ids_b[b]

            def grp(g):
                idg = iv[pl.ds(g * 16, 16)]
                for j in range(16):
                    sj = idg[j]
                    row = g * 16 + j
                    for k8 in range(CBLK // 16):
                        plsc.addupdate(acc.at[sj, pl.ds(k8 * 16, 16)],
                                       rv[row, pl.ds(k8 * 16, 16)])

            plsc.parallel_loop(0, TILE_ROWS // 16)(grp)

        def zr(g, carry):
            for k8 in range(CBLK // 16):
                acc[g, pl.ds(k8 * 16, 16)] = jnp.zeros((16,), jnp.float32)
            return carry

        lax.fori_loop(0, G, zr, 0)

        start(0, rg)
        start(1, NRG + rg)

        def outer(i2, carry):
            for b in range(2):
                i = i2 * 2 + b
                wait(b)
                compute(b)

                @pl.when(i + 2 < CPW)
                def _():
                    start(b, (i + 2) * NRG + rg)

            return carry

        lax.fori_loop(0, CPW // 2, outer, 0)

        @pl.when(rg == 0)
        def _last():
            start(0, LAST_TILE)
            wait(0)
            compute(0)

        pltpu.sync_copy(acc, part_hbm.at[rg, :, pl.ds(col0, CBLK)])
        plsc.subcore_barrier()

        row0 = sid * 32
        colc = cid * HC
        pltpu.sync_copy(part_hbm.at[0, pl.ds(row0, 32), pl.ds(colc, HC)], av)

        def comb(r2, carry):
            pltpu.sync_copy(part_hbm.at[r2, pl.ds(row0, 32), pl.ds(colc, HC)], pv)

            def crow(r, c2):
                for kk in range(HC // 16):
                    av[r, pl.ds(kk * 16, 16)] = (av[r, pl.ds(kk * 16, 16)]
                                                 + pv[r, pl.ds(kk * 16, 16)])
                return c2

            lax.fori_loop(0, 32, crow, 0)
            return carry

        lax.fori_loop(1, NRG, comb, 0)
        pltpu.sync_copy(av, out_hbm.at[pl.ds(row0, 32), pl.ds(colc, HC)])

    out, _ = k(gated, ids)
    return out


def kernel(node_features, node_to_graph_id, W_proj, b_proj, W_gate, b_gate):
    ids = node_to_graph_id.astype(jnp.int32)
    gated = _gated_mlp(node_features, W_proj, b_proj.reshape(1, H),
                       W_gate, b_gate.reshape(1, H))
    return _seg_sum_sc(gated, ids)


# speedup vs baseline: 1.9383x; 1.2953x over previous
import functools

import jax
import jax.numpy as jnp
from jax import lax
from jax.experimental import pallas as pl
from jax.experimental.pallas import tpu as pltpu
from jax.experimental.pallas import tpu_sc as plsc

N, D, H, G = 50000, 512, 512, 512
R = 2000
GRID = N // R

TILE_ROWS = 80
N_TILES = N // TILE_ROWS
CBLK = 128
NRG = 8
CPW = N_TILES // NRG
LAST_TILE = CPW * NRG
HC = H // 2


def _mlp_body(x_ref, wp_ref, bp_ref, wg_ref, bg_ref, out_ref):
    x = x_ref[...]
    proj = jnp.dot(x, wp_ref[...], preferred_element_type=jnp.float32) + bp_ref[...]
    gate_l = jnp.dot(x, wg_ref[...], preferred_element_type=jnp.float32) + bg_ref[...]
    out_ref[...] = jax.nn.sigmoid(gate_l) * proj


def _gated_mlp(x, wp, bp, wg, bg):
    return pl.pallas_call(
        _mlp_body,
        grid=(GRID,),
        in_specs=[
            pl.BlockSpec((R, D), lambda i: (i, 0)),
            pl.BlockSpec((D, H), lambda i: (0, 0)),
            pl.BlockSpec((1, H), lambda i: (0, 0)),
            pl.BlockSpec((D, H), lambda i: (0, 0)),
            pl.BlockSpec((1, H), lambda i: (0, 0)),
        ],
        out_specs=pl.BlockSpec((R, H), lambda i: (i, 0)),
        out_shape=jax.ShapeDtypeStruct((N, H), jnp.float32),
    )(x, wp, bp, wg, bg)


def _seg_sum_sc(gated, ids):
    mesh = plsc.VectorSubcoreMesh(core_axis_name="c", subcore_axis_name="s")

    @functools.partial(
        pl.kernel,
        mesh=mesh,
        out_type=(jax.ShapeDtypeStruct((G, H), jnp.float32),
                  jax.ShapeDtypeStruct((NRG, G, H), jnp.float32)),
        scratch_types=[
            pltpu.VMEM((G, CBLK), jnp.float32),
            pltpu.VMEM((TILE_ROWS, CBLK), jnp.float32),
            pltpu.VMEM((TILE_ROWS, CBLK), jnp.float32),
            pltpu.VMEM((TILE_ROWS,), jnp.int32),
            pltpu.VMEM((TILE_ROWS,), jnp.int32),
            pltpu.VMEM((32, HC), jnp.float32),
            pltpu.VMEM((32, HC), jnp.float32),
            pltpu.SemaphoreType.DMA,
            pltpu.SemaphoreType.DMA,
            pltpu.SemaphoreType.DMA,
            pltpu.SemaphoreType.DMA,
        ],
    )
    def k(gated_hbm, ids_hbm, out_hbm, part_hbm, acc, rows0, rows1,
          ids0, ids1, av, pv, sr0, sr1, si0, si1):
        cid = lax.axis_index("c")
        sid = lax.axis_index("s")
        rg = sid // 2
        col0 = cid * HC + (sid % 2) * CBLK

        rows_b = (rows0, rows1)
        ids_b = (ids0, ids1)
        sr = (sr0, sr1)
        si = (si0, si1)

        def start(b, t):
            base = t * TILE_ROWS
            pltpu.async_copy(
                gated_hbm.at[pl.ds(base, TILE_ROWS), pl.ds(col0, CBLK)],
                rows_b[b], sr[b])
            pltpu.async_copy(ids_hbm.at[pl.ds(base, TILE_ROWS)], ids_b[b], si[b])

        def wait(b):
            pltpu.make_async_copy(
                gated_hbm.at[pl.ds(0, TILE_ROWS), pl.ds(col0, CBLK)],
                rows_b[b], sr[b]).wait()
            pltpu.make_async_copy(ids_hbm.at[pl.ds(0, TILE_ROWS)],
                                  ids_b[b], si[b]).wait()

        def compute(b):
            rv, iv = rows_b[b], ids_b[b]

            def grp(g):
                idg = iv[pl.ds(g * 16, 16)]
                first = idg[0]
                last = idg[15]

                @pl.when(first == last)
                def _uniform():
                    for k8 in range(CBLK // 16):
                        cs = pl.ds(k8 * 16, 16)
                        v = [rv[g * 16 + j, cs] for j in range(16)]
                        while len(v) > 1:
                            v = [v[i] + v[i + 1] for i in range(0, len(v), 2)]
                        plsc.addupdate(acc.at[first, cs], v[0])

                @pl.when(first != last)
                def _mixed():
                    for j in range(16):
                        sj = idg[j]
                        row = g * 16 + j
                        for k8 in range(CBLK // 16):
                            plsc.addupdate(acc.at[sj, pl.ds(k8 * 16, 16)],
                                           rv[row, pl.ds(k8 * 16, 16)])

            plsc.parallel_loop(0, TILE_ROWS // 16)(grp)

        def zr(g, carry):
            for k8 in range(CBLK // 16):
                acc[g, pl.ds(k8 * 16, 16)] = jnp.zeros((16,), jnp.float32)
            return carry

        lax.fori_loop(0, G, zr, 0)

        start(0, rg)
        start(1, NRG + rg)

        def outer(i2, carry):
            for b in range(2):
                i = i2 * 2 + b
                wait(b)
                compute(b)

                @pl.when(i + 2 < CPW)
                def _():
                    start(b, (i + 2) * NRG + rg)

            return carry

        lax.fori_loop(0, CPW // 2, outer, 0)

        @pl.when(rg == 0)
        def _last():
            start(0, LAST_TILE)
            wait(0)
            compute(0)

        pltpu.sync_copy(acc, part_hbm.at[rg, :, pl.ds(col0, CBLK)])
        plsc.subcore_barrier()

        row0 = sid * 32
        colc = cid * HC
        pltpu.sync_copy(part_hbm.at[0, pl.ds(row0, 32), pl.ds(colc, HC)], av)

        def comb(r2, carry):
            pltpu.sync_copy(part_hbm.at[r2, pl.ds(row0, 32), pl.ds(colc, HC)], pv)

            def crow(r, c2):
                for kk in range(HC // 16):
                    av[r, pl.ds(kk * 16, 16)] = (av[r, pl.ds(kk * 16, 16)]
                                                 + pv[r, pl.ds(kk * 16, 16)])
                return c2

            lax.fori_loop(0, 32, crow, 0)
            return carry

        lax.fori_loop(1, NRG, comb, 0)
        pltpu.sync_copy(av, out_hbm.at[pl.ds(row0, 32), pl.ds(colc, HC)])

    out, _ = k(gated, ids)
    return out


def kernel(node_features, node_to_graph_id, W_proj, b_proj, W_gate, b_gate):
    ids = node_to_graph_id.astype(jnp.int32)
    gated = _gated_mlp(node_features, W_proj, b_proj.reshape(1, H),
                       W_gate, b_gate.reshape(1, H))
    return _seg_sum_sc(gated, ids)
